# Initial kernel scaffold; baseline (speedup 1.0000x reference)
import jax
import jax.numpy as jnp
from jax.experimental import pallas as pl


def kernel(tokens, table):
    raise NotImplementedError("write your pallas kernel here")


# speedup vs baseline: 3.7674x; 3.7674x over previous
import functools
import math

import jax
import jax.numpy as jnp
from jax import lax
from jax.experimental import pallas as pl
from jax.experimental.pallas import tpu as pltpu
from jax.experimental.pallas import tpu_sc as plsc

VOCAB = 100000
EMB = 64
SCALE = math.sqrt(EMB)

NC = 2
NS = 16
NW = NC * NS

B = 4096 * 200
B_PER_W = B // NW
C = 128
NCH = B_PER_W // C
NBUF = 4


def _scale_body(t_ref, o_ref):
    o_ref[...] = t_ref[...] * SCALE


@jax.jit
def _scale_table(table):
    return pl.pallas_call(
        _scale_body,
        grid=(100,),
        in_specs=[pl.BlockSpec((VOCAB // 100, EMB), lambda i: (i, 0))],
        out_specs=pl.BlockSpec((VOCAB // 100, EMB), lambda i: (i, 0)),
        out_shape=jax.ShapeDtypeStruct((VOCAB, EMB), jnp.float32),
    )(table)


def _gather_body(table_hbm, idx_hbm, out_hbm, idx_v, rows_v, gsem, wsem):
    wid = lax.axis_index("s") * NC + lax.axis_index("c")
    base = wid * B_PER_W

    pltpu.sync_copy(idx_hbm.at[wid], idx_v)

    for b in range(NBUF):
        pltpu.async_copy(table_hbm.at[idx_v.at[b]], rows_v.at[b], gsem.at[b])

    @pl.loop(0, NCH - NBUF, step=NBUF)
    def _main(g):
        for b in range(NBUF):
            j = g + b
            pltpu.make_async_copy(
                table_hbm.at[idx_v.at[b]], rows_v.at[b], gsem.at[b]
            ).wait()
            dst = out_hbm.at[pl.ds(base + j * C, C)]
            pltpu.async_copy(rows_v.at[b], dst, wsem.at[b])
            pltpu.make_async_copy(rows_v.at[b], dst, wsem.at[b]).wait()
            pltpu.async_copy(
                table_hbm.at[idx_v.at[j + NBUF]], rows_v.at[b], gsem.at[b]
            )

    for b in range(NBUF):
        j = NCH - NBUF + b
        pltpu.make_async_copy(
            table_hbm.at[idx_v.at[b]], rows_v.at[b], gsem.at[b]
        ).wait()
        dst = out_hbm.at[pl.ds(base + j * C, C)]
        pltpu.async_copy(rows_v.at[b], dst, wsem.at[b])
    for b in range(NBUF):
        j = NCH - NBUF + b
        dst = out_hbm.at[pl.ds(base + j * C, C)]
        pltpu.make_async_copy(rows_v.at[b], dst, wsem.at[b]).wait()


@jax.jit
def _gather(table_scaled, idx):
    mesh = plsc.VectorSubcoreMesh(core_axis_name="c", subcore_axis_name="s")
    return pl.kernel(
        _gather_body,
        out_type=jax.ShapeDtypeStruct((B, EMB), jnp.float32),
        mesh=mesh,
        scratch_types=[
            pltpu.VMEM((NCH, C), jnp.int32),
            pltpu.VMEM((NBUF, C, EMB), jnp.float32),
            pltpu.SemaphoreType.DMA((NBUF,)),
            pltpu.SemaphoreType.DMA((NBUF,)),
        ],
        compiler_params=pltpu.CompilerParams(use_tc_tiling_on_sc=False),
    )(table_scaled, idx)


def kernel(tokens, table):
    table_scaled = _scale_table(table)
    idx = tokens.reshape(NW, NCH, C)
    out = _gather(table_scaled, idx)
    return out.reshape(tokens.shape[0], tokens.shape[1], EMB)


# speedup vs baseline: 5.8710x; 1.5584x over previous
import math

import jax
import jax.numpy as jnp
from jax import lax
from jax.experimental import pallas as pl
from jax.experimental.pallas import tpu as pltpu
from jax.experimental.pallas import tpu_sc as plsc

VOCAB = 100000
EMB = 64
SCALE = math.sqrt(EMB)

NC = 2
NS = 16
NW = NC * NS
L = 16

NB = 4096
NT = 200
BU = 128
NU = NT * (NB // BU)
U_PER_W = NU // NW
EG = EMB // 8
NRB = 4


def _gather_body(
    table_hbm, idx_hbm, out_hbm, idx_v, rows_v, r65_v, tbuf_v, gsem, wsem
):
    wid = lax.axis_index("s") * NC + lax.axis_index("c")
    u0 = wid * U_PER_W

    pltpu.sync_copy(idx_hbm.at[pl.ds(u0, U_PER_W)], idx_v)

    viota = lax.iota(jnp.int32, L)

    def fire_gather(j, buf):
        pltpu.async_copy(
            table_hbm.at[idx_v.at[j]], rows_v.at[buf], gsem.at[buf]
        )

    def wait_gather(buf):
        pltpu.make_async_copy(
            table_hbm.at[idx_v.at[0]], rows_v.at[buf], gsem.at[buf]
        ).wait()

    def transpose_unit(rb, tb):
        @plsc.parallel_loop(0, BU)
        def _r(r):
            for c0 in range(EMB // L):
                v = rows_v[rb, r, pl.ds(c0 * L, L)]
                r65_v[r, pl.ds(c0 * L, L)] = v * SCALE

        @plsc.parallel_loop(0, 8)
        def _l0(l0):
            row_ids = viota + l0 * 16
            dst0 = l0 * 16
            for e in range(EMB):
                col_ids = lax.broadcast(e, (L,))
                vals = plsc.load_gather(r65_v, [row_ids, col_ids])
                tbuf_v[tb, e >> 3, e & 7, pl.ds(dst0, L)] = vals

    def dst_slice(j):
        u = u0 + j
        t = u >> 5
        bg = u & 31
        return out_hbm.at[t, :, bg]

    for rb in range(NRB):
        fire_gather(rb, rb)

    @pl.loop(0, U_PER_W, step=NRB)
    def _main(g):
        for rb in range(NRB):
            j = g + rb
            tb = rb & 1
            wait_gather(rb)

            @pl.when(j >= 2)
            def _():
                pltpu.make_async_copy(
                    tbuf_v.at[tb], dst_slice(j - 2), wsem.at[tb]
                ).wait()

            transpose_unit(rb, tb)

            @pl.when(j + NRB < U_PER_W)
            def _():
                fire_gather(j + NRB, rb)

            pltpu.async_copy(tbuf_v.at[tb], dst_slice(j), wsem.at[tb])

    for tb in range(2):
        j = U_PER_W - 2 + tb
        pltpu.make_async_copy(tbuf_v.at[tb], dst_slice(j), wsem.at[tb]).wait()


def _impl(tokens, table):
    mesh = plsc.VectorSubcoreMesh(core_axis_name="c", subcore_axis_name="s")
    idx = tokens.T.reshape(NU, 128)
    out5 = pl.kernel(
        _gather_body,
        out_type=jax.ShapeDtypeStruct((NT, EG, NB // 128, 8, 128), jnp.float32),
        mesh=mesh,
        scratch_types=[
            pltpu.VMEM((U_PER_W, 128), jnp.int32),
            pltpu.VMEM((NRB, BU, EMB), jnp.float32),
            pltpu.VMEM((BU, EMB + 1), jnp.float32),
            pltpu.VMEM((2, EG, 8, 128), jnp.float32),
            pltpu.SemaphoreType.DMA((NRB,)),
            pltpu.SemaphoreType.DMA((2,)),
        ],
        compiler_params=pltpu.CompilerParams(
            use_tc_tiling_on_sc=False, needs_layout_passes=False
        ),
    )(table, idx)
    return out5.transpose(2, 4, 0, 1, 3).reshape(NB, NT, EMB)


def kernel(tokens, table):
    return _impl(tokens, table)


# speedup vs baseline: 11.0153x; 1.8762x over previous
import math

import jax
import jax.numpy as jnp
from jax import lax
from jax.experimental import pallas as pl
from jax.experimental.pallas import tpu as pltpu
from jax.experimental.pallas import tpu_sc as plsc

VOCAB = 100000
EMB = 64
SCALE = math.sqrt(EMB)

NC = 2
NS = 16
NW = NC * NS
L = 16

NB = 4096
NT = 200
BU = 128
NU = NT * (NB // BU)
U_PER_W = NU // NW
EG = EMB // 8
NRB = 4


def _gather_body(
    table_hbm, idx_hbm, out_hbm, idx_v, rows_v, r65_v, tbuf_v, gsem, wsem
):
    wid = lax.axis_index("s") * NC + lax.axis_index("c")
    u0 = wid * U_PER_W

    pltpu.sync_copy(idx_hbm.at[pl.ds(u0, U_PER_W)], idx_v)

    viota = lax.iota(jnp.int32, L)

    def fire_gather(j, buf):
        pltpu.async_copy(
            table_hbm.at[idx_v.at[j]], rows_v.at[buf], gsem.at[buf]
        )

    def wait_gather(buf):
        pltpu.make_async_copy(
            table_hbm.at[idx_v.at[0]], rows_v.at[buf], gsem.at[buf]
        ).wait()

    def transpose_unit(rb, tb):
        @plsc.parallel_loop(0, BU)
        def _r(r):
            for c0 in range(EMB // L):
                v = rows_v[rb, r, pl.ds(c0 * L, L)]
                r65_v[r, pl.ds(c0 * L, L)] = v * SCALE

        @plsc.parallel_loop(0, 64, unroll=2)
        def _i(i):
            l0 = i >> 3
            eg = i & 7
            row_ids = viota + l0 * 16
            dst0 = l0 * 16
            for es in range(8):
                col_ids = lax.broadcast(eg * 8 + es, (L,))
                vals = plsc.load_gather(r65_v, [row_ids, col_ids])
                tbuf_v[tb, eg, es, pl.ds(dst0, L)] = vals

    def dst_slice(j):
        u = u0 + j
        t = u >> 5
        bg = u & 31
        return out_hbm.at[t, :, bg]

    for rb in range(NRB):
        fire_gather(rb, rb)

    @pl.loop(0, U_PER_W, step=NRB)
    def _main(g):
        for rb in range(NRB):
            j = g + rb
            tb = rb & 1
            wait_gather(rb)

            @pl.when(j >= 2)
            def _():
                pltpu.make_async_copy(
                    tbuf_v.at[tb], dst_slice(j - 2), wsem.at[tb]
                ).wait()

            transpose_unit(rb, tb)

            @pl.when(j + NRB < U_PER_W)
            def _():
                fire_gather(j + NRB, rb)

            pltpu.async_copy(tbuf_v.at[tb], dst_slice(j), wsem.at[tb])

    for tb in range(2):
        j = U_PER_W - 2 + tb
        pltpu.make_async_copy(tbuf_v.at[tb], dst_slice(j), wsem.at[tb]).wait()


def _impl(tokens, table):
    mesh = plsc.VectorSubcoreMesh(core_axis_name="c", subcore_axis_name="s")
    idx = tokens.T.reshape(NU, 128)
    out5 = pl.kernel(
        _gather_body,
        out_type=jax.ShapeDtypeStruct((NT, EG, NB // 128, 8, 128), jnp.float32),
        mesh=mesh,
        scratch_types=[
            pltpu.VMEM((U_PER_W, 128), jnp.int32),
            pltpu.VMEM((NRB, BU, EMB), jnp.float32),
            pltpu.VMEM((BU, EMB + 1), jnp.float32),
            pltpu.VMEM((2, EG, 8, 128), jnp.float32),
            pltpu.SemaphoreType.DMA((NRB,)),
            pltpu.SemaphoreType.DMA((2,)),
        ],
        compiler_params=pltpu.CompilerParams(
            use_tc_tiling_on_sc=False, needs_layout_passes=False
        ),
    )(table, idx)
    return out5.transpose(2, 4, 0, 1, 3).reshape(NB, NT, EMB)


def kernel(tokens, table):
    return _impl(tokens, table)


# speedup vs baseline: 12.4324x; 1.1286x over previous
import math

import jax
import jax.numpy as jnp
from jax import lax
from jax.experimental import pallas as pl
from jax.experimental.pallas import tpu as pltpu
from jax.experimental.pallas import tpu_sc as plsc

VOCAB = 100000
EMB = 64
SCALE = math.sqrt(EMB)

NC = 2
NS = 16
NW = NC * NS
L = 16

NB = 4096
NT = 200
BU = 128
NU = NT * (NB // BU)
U_PER_W = NU // NW
EG = EMB // 8
NRB = 4


def _gather_body(
    table_hbm, idx_hbm, out_hbm, idx_v, rows_v, r65_v, tbuf_v, gsem, wsem
):
    wid = lax.axis_index("s") * NC + lax.axis_index("c")
    u0 = wid * U_PER_W

    pltpu.sync_copy(idx_hbm.at[pl.ds(u0, U_PER_W)], idx_v)

    viota = lax.iota(jnp.int32, L)

    def fire_gather(j, buf):
        pltpu.async_copy(
            table_hbm.at[idx_v.at[j]], rows_v.at[buf], gsem.at[buf]
        )

    def wait_gather(buf):
        pltpu.make_async_copy(
            table_hbm.at[idx_v.at[0]], rows_v.at[buf], gsem.at[buf]
        ).wait()

    def transpose_unit(rb, tb):
        @plsc.parallel_loop(0, BU, unroll=2)
        def _r(r):
            for c0 in range(EMB // L):
                v = rows_v[rb, r, pl.ds(c0 * L, L)]
                r65_v[r, pl.ds(c0 * L, L)] = v

        @plsc.parallel_loop(0, 64, unroll=2)
        def _i(i):
            l0 = i >> 3
            eg = i & 7
            row_ids = viota + l0 * 16
            dst0 = l0 * 16
            for es in range(8):
                col_ids = lax.broadcast(eg * 8 + es, (L,))
                vals = plsc.load_gather(r65_v, [row_ids, col_ids])
                tbuf_v[tb, eg, es, pl.ds(dst0, L)] = vals * SCALE

    def dst_slice(j):
        u = u0 + j
        t = u >> 5
        bg = u & 31
        return out_hbm.at[t, :, bg]

    for rb in range(NRB):
        fire_gather(rb, rb)

    @pl.loop(0, U_PER_W, step=NRB)
    def _main(g):
        for rb in range(NRB):
            j = g + rb
            tb = rb & 1
            wait_gather(rb)

            @pl.when(j >= 2)
            def _():
                pltpu.make_async_copy(
                    tbuf_v.at[tb], dst_slice(j - 2), wsem.at[tb]
                ).wait()

            transpose_unit(rb, tb)

            @pl.when(j + NRB < U_PER_W)
            def _():
                fire_gather(j + NRB, rb)

            pltpu.async_copy(tbuf_v.at[tb], dst_slice(j), wsem.at[tb])

    for tb in range(2):
        j = U_PER_W - 2 + tb
        pltpu.make_async_copy(tbuf_v.at[tb], dst_slice(j), wsem.at[tb]).wait()


def _impl(tokens, table):
    mesh = plsc.VectorSubcoreMesh(core_axis_name="c", subcore_axis_name="s")
    idx = tokens.T.reshape(NU, 128)
    out5 = pl.kernel(
        _gather_body,
        out_type=jax.ShapeDtypeStruct((NT, EG, NB // 128, 8, 128), jnp.float32),
        mesh=mesh,
        scratch_types=[
            pltpu.VMEM((U_PER_W, 128), jnp.int32),
            pltpu.VMEM((NRB, BU, EMB), jnp.float32),
            pltpu.VMEM((BU, EMB + 1), jnp.float32),
            pltpu.VMEM((2, EG, 8, 128), jnp.float32),
            pltpu.SemaphoreType.DMA((NRB,)),
            pltpu.SemaphoreType.DMA((2,)),
        ],
        compiler_params=pltpu.CompilerParams(
            use_tc_tiling_on_sc=False, needs_layout_passes=False
        ),
    )(table, idx)
    return out5.transpose(2, 4, 0, 1, 3).reshape(NB, NT, EMB)


def kernel(tokens, table):
    return _impl(tokens, table)
